# baseline (speedup 1.0000x reference)
import functools

import jax
import jax.numpy as jnp
from jax import lax
from jax.experimental import pallas as pl
from jax.experimental.pallas import tpu as pltpu
from jax.experimental.pallas import tpu_sc as plsc

B = 1024
L = 200
E = 64
F = 32
U = 100
NCLS = 1000
UP = 128
G3 = 3 * UP

NC = 2
NS = 16
NW = NC * NS
R = B * L
PER_W = R // NW
CHUNK = 128
NCH = PER_W // CHUNK
KF = 10
ROUNDS = NCH // KF


def _sc_gather_body(table_hbm, idx_hbm, out_hbm, idx_v, bufs, gsem, wsem):
    c = lax.axis_index("c")
    s = lax.axis_index("s")
    wid = s * NC + c
    base = wid * PER_W
    pltpu.sync_copy(idx_hbm.at[wid], idx_v)

    def round_body(r, _):
        ghandles = []
        for j in range(KF):
            ch = r * KF + j
            h = pltpu.async_copy(table_hbm.at[idx_v.at[ch]], bufs.at[j], gsem)
            ghandles.append((h, ch))
        whandles = []
        for j in range(KF):
            h, ch = ghandles[j]
            h.wait()
            wh = pltpu.async_copy(
                bufs.at[j], out_hbm.at[pl.ds(base + ch * CHUNK, CHUNK)], wsem)
            whandles.append(wh)
        for wh in whandles:
            wh.wait()
        return _

    lax.fori_loop(0, ROUNDS, round_body, 0)


_sc_gather_fn = None


def _sc_gather(table, idx):
    global _sc_gather_fn
    if _sc_gather_fn is None:
        _sc_gather_fn = pl.kernel(
            _sc_gather_body,
            mesh=plsc.VectorSubcoreMesh(core_axis_name="c",
                                        subcore_axis_name="s"),
            out_type=jax.ShapeDtypeStruct((R, E), jnp.float32),
            compiler_params=pltpu.CompilerParams(use_tc_tiling_on_sc=False),
            scratch_types=[
                pltpu.VMEM((NCH, CHUNK), jnp.int32),
                pltpu.VMEM((KF, CHUNK, E), jnp.float32),
                pltpu.SemaphoreType.DMA,
                pltpu.SemaphoreType.DMA,
            ],
        )
    return _sc_gather_fn(table, idx)


def _tc_body(x_ref, wb_ref, cb_ref, gk_ref, gr_ref, bi_ref, br_ref,
             dw_ref, db_ref, o_ref, p_ref):
    cb = x_ref.shape[1]
    f32 = jnp.float32
    xf = x_ref[...].reshape(U * cb, 2 * E)
    y6 = jnp.dot(xf, wb_ref[...], preferred_element_type=f32)
    ye0, yo0 = y6[:, 0:F], y6[:, F:2 * F]
    ye1, yo1 = y6[:, 2 * F:3 * F], y6[:, 3 * F:4 * F]
    ye2, yo2 = y6[:, 4 * F:5 * F], y6[:, 5 * F:6 * F]
    zrow = jnp.zeros((cb, F), f32)
    yo0s = jnp.concatenate([zrow, yo0[:-cb]], axis=0)
    ye2s = jnp.concatenate([ye2[cb:], zrow], axis=0)
    bias = cb_ref[...]
    c_even = jnp.maximum(ye1 + yo0s + yo2 + bias, 0.0)
    c_odd = jnp.maximum(yo1 + ye0 + ye2s + bias, 0.0)
    p_ref[...] = jnp.maximum(c_even, c_odd)

    gk = gk_ref[...]
    gr = gr_ref[...]
    bi = bi_ref[...]
    br = br_ref[...]

    def step(t, h):
        xt = p_ref[pl.ds(t * cb, cb), :]
        xg = jnp.dot(xt, gk, preferred_element_type=f32) + bi
        hg = jnp.dot(h, gr, preferred_element_type=f32) + br
        xz, xr, xh = xg[:, :UP], xg[:, UP:2 * UP], xg[:, 2 * UP:]
        hz, hr, hn = hg[:, :UP], hg[:, UP:2 * UP], hg[:, 2 * UP:]
        z = 1.0 / (1.0 + jnp.exp(-(xz + hz)))
        r = 1.0 / (1.0 + jnp.exp(-(xr + hr)))
        n = jnp.tanh(xh + r * hn)
        return z * h + (1.0 - z) * n

    h = lax.fori_loop(0, U, step, jnp.zeros((cb, UP), f32))
    logits = jnp.dot(h, dw_ref[...], preferred_element_type=f32) + db_ref[...]
    m = jnp.max(logits, axis=-1, keepdims=True)
    e = jnp.exp(logits - m)
    o_ref[...] = e / jnp.sum(e, axis=-1, keepdims=True)


def _pad_gates(w):
    parts = []
    for g in range(3):
        blk = w[..., g * U:(g + 1) * U]
        pad = [(0, 0)] * (w.ndim - 1) + [(0, UP - U)]
        parts.append(jnp.pad(blk, pad))
    return jnp.concatenate(parts, axis=-1)


def kernel(inputs, table, conv_w, conv_b, gru_k, gru_r, gru_b, dense_w, dense_b):
    idxp = (inputs.astype(jnp.int32).reshape(B, U, 2)
            .transpose(1, 0, 2).reshape(NW, NCH, CHUNK))
    x = _sc_gather(table, idxp)
    x3 = x.reshape(U, B, 2 * E)

    wb = jnp.zeros((2 * E, 6 * F), jnp.float32)
    for k in range(3):
        wb = wb.at[0:E, 2 * F * k:2 * F * k + F].set(conv_w[k])
        wb = wb.at[E:2 * E, 2 * F * k + F:2 * F * k + 2 * F].set(conv_w[k])
    cbias = conv_b.reshape(1, F)
    gk = _pad_gates(gru_k)
    gr = jnp.pad(_pad_gates(gru_r), ((0, UP - U), (0, 0)))
    bi = _pad_gates(gru_b[0]).reshape(1, G3)
    br = _pad_gates(gru_b[1]).reshape(1, G3)
    dw = jnp.pad(dense_w, ((0, UP - U), (0, 0)))
    db = dense_b.reshape(1, NCLS)

    CB = 128
    grid = (B // CB,)
    out = pl.pallas_call(
        _tc_body,
        grid=grid,
        in_specs=[
            pl.BlockSpec((U, CB, 2 * E), lambda i: (0, i, 0)),
            pl.BlockSpec((2 * E, 6 * F), lambda i: (0, 0)),
            pl.BlockSpec((1, F), lambda i: (0, 0)),
            pl.BlockSpec((F, G3), lambda i: (0, 0)),
            pl.BlockSpec((UP, G3), lambda i: (0, 0)),
            pl.BlockSpec((1, G3), lambda i: (0, 0)),
            pl.BlockSpec((1, G3), lambda i: (0, 0)),
            pl.BlockSpec((UP, NCLS), lambda i: (0, 0)),
            pl.BlockSpec((1, NCLS), lambda i: (0, 0)),
        ],
        out_specs=pl.BlockSpec((CB, NCLS), lambda i: (i, 0)),
        out_shape=jax.ShapeDtypeStruct((B, NCLS), jnp.float32),
        scratch_shapes=[pltpu.VMEM((U * CB, F), jnp.float32)],
    )(x3, wb, cbias, gk, gr, bi, br, dw, db)
    return out


# speedup vs baseline: 1.1273x; 1.1273x over previous
import functools

import jax
import jax.numpy as jnp
from jax import lax
from jax.experimental import pallas as pl
from jax.experimental.pallas import tpu as pltpu
from jax.experimental.pallas import tpu_sc as plsc

B = 1024
L = 200
E = 64
F = 32
U = 100
NCLS = 1000
UP = 128
G3 = 3 * UP

NC = 2
NS = 16
NW = NC * NS
R = B * L
PER_W = R // NW
CHUNK = 128
NCH = PER_W // CHUNK
KF = 10
ROUNDS = NCH // KF


def _sc_gather_body(table_hbm, idx_hbm, out_hbm, idx_v, bufs, gsem, wsem):
    c = lax.axis_index("c")
    s = lax.axis_index("s")
    wid = s * NC + c
    base = wid * PER_W
    pltpu.sync_copy(idx_hbm.at[wid], idx_v)

    def round_body(r, _):
        ghandles = []
        for j in range(KF):
            ch = r * KF + j
            h = pltpu.async_copy(table_hbm.at[idx_v.at[ch]], bufs.at[j], gsem)
            ghandles.append((h, ch))
        whandles = []
        for j in range(KF):
            h, ch = ghandles[j]
            h.wait()
            wh = pltpu.async_copy(
                bufs.at[j], out_hbm.at[pl.ds(base + ch * CHUNK, CHUNK)], wsem)
            whandles.append(wh)
        for wh in whandles:
            wh.wait()
        return _

    lax.fori_loop(0, ROUNDS, round_body, 0)


_sc_gather_fn = None


def _sc_gather(table, idx):
    global _sc_gather_fn
    if _sc_gather_fn is None:
        _sc_gather_fn = pl.kernel(
            _sc_gather_body,
            mesh=plsc.VectorSubcoreMesh(core_axis_name="c",
                                        subcore_axis_name="s"),
            out_type=jax.ShapeDtypeStruct((R, E), jnp.float32),
            compiler_params=pltpu.CompilerParams(use_tc_tiling_on_sc=False),
            scratch_types=[
                pltpu.VMEM((NCH, CHUNK), jnp.int32),
                pltpu.VMEM((KF, CHUNK, E), jnp.float32),
                pltpu.SemaphoreType.DMA,
                pltpu.SemaphoreType.DMA,
            ],
        )
    return _sc_gather_fn(table, idx)


def _tc_body(x_ref, wb_ref, cb_ref, gk_ref, gr_ref, bi_ref, br_ref,
             dw_ref, db_ref, o_ref, y6_ref, h_ref):
    i = pl.program_id(0)
    f32 = jnp.float32
    bf16 = jnp.bfloat16

    xf = x_ref[0].astype(bf16)
    y6 = jnp.dot(xf, wb_ref[...], preferred_element_type=f32)
    y6 = jnp.where(i < U, y6, 0.0)

    @pl.when(i == 0)
    def _init():
        y6_ref[1] = jnp.zeros((B, 6 * F), f32)
        h_ref[...] = jnp.zeros((B, UP), f32)

    @pl.when(i > 0)
    def _step():
        y6_m1 = y6_ref[(i + 1) % 2]
        y6_m2 = y6_ref[i % 2]
        bias = cb_ref[...]
        c_even = jnp.maximum(
            y6_m1[:, 2 * F:3 * F] + y6_m2[:, F:2 * F] + y6_m1[:, 5 * F:6 * F]
            + bias, 0.0)
        c_odd = jnp.maximum(
            y6_m1[:, 0:F] + y6_m1[:, 3 * F:4 * F] + y6[:, 4 * F:5 * F]
            + bias, 0.0)
        xt = jnp.maximum(c_even, c_odd).astype(bf16)
        h = h_ref[...]
        xg = jnp.dot(xt, gk_ref[...], preferred_element_type=f32) + bi_ref[...]
        hg = (jnp.dot(h.astype(bf16), gr_ref[...], preferred_element_type=f32)
              + br_ref[...])
        xz, xr, xh = xg[:, :UP], xg[:, UP:2 * UP], xg[:, 2 * UP:]
        hz, hr, hn = hg[:, :UP], hg[:, UP:2 * UP], hg[:, 2 * UP:]
        z = 1.0 / (1.0 + jnp.exp(-(xz + hz)))
        r = 1.0 / (1.0 + jnp.exp(-(xr + hr)))
        n = jnp.tanh(xh + r * hn)
        h = z * h + (1.0 - z) * n
        h_ref[...] = h

        @pl.when(i == U)
        def _final():
            logits = (jnp.dot(h.astype(bf16), dw_ref[...],
                              preferred_element_type=f32) + db_ref[...])
            m = jnp.max(logits, axis=-1, keepdims=True)
            e = jnp.exp(logits - m)
            o_ref[...] = e / jnp.sum(e, axis=-1, keepdims=True)

    y6_ref[i % 2] = y6


def _pad_gates(w):
    parts = []
    for g in range(3):
        blk = w[..., g * U:(g + 1) * U]
        pad = [(0, 0)] * (w.ndim - 1) + [(0, UP - U)]
        parts.append(jnp.pad(blk, pad))
    return jnp.concatenate(parts, axis=-1)


def kernel(inputs, table, conv_w, conv_b, gru_k, gru_r, gru_b, dense_w, dense_b):
    idxp = (inputs.astype(jnp.int32).reshape(B, U, 2)
            .transpose(1, 0, 2).reshape(NW, NCH, CHUNK))
    x = _sc_gather(table, idxp)
    x3 = x.reshape(U, B, 2 * E)

    wb = jnp.zeros((2 * E, 6 * F), jnp.float32)
    for k in range(3):
        wb = wb.at[0:E, 2 * F * k:2 * F * k + F].set(conv_w[k])
        wb = wb.at[E:2 * E, 2 * F * k + F:2 * F * k + 2 * F].set(conv_w[k])
    cbias = conv_b.reshape(1, F)
    gk = _pad_gates(gru_k)
    gr = jnp.pad(_pad_gates(gru_r), ((0, UP - U), (0, 0)))
    bi = _pad_gates(gru_b[0]).reshape(1, G3)
    br = _pad_gates(gru_b[1]).reshape(1, G3)
    dw = jnp.pad(dense_w, ((0, UP - U), (0, 0)))
    db = dense_b.reshape(1, NCLS)

    bf16 = jnp.bfloat16
    grid = (U + 1,)
    out = pl.pallas_call(
        _tc_body,
        grid=grid,
        in_specs=[
            pl.BlockSpec((1, B, 2 * E), lambda i: (jnp.minimum(i, U - 1), 0, 0)),
            pl.BlockSpec((2 * E, 6 * F), lambda i: (0, 0)),
            pl.BlockSpec((1, F), lambda i: (0, 0)),
            pl.BlockSpec((F, G3), lambda i: (0, 0)),
            pl.BlockSpec((UP, G3), lambda i: (0, 0)),
            pl.BlockSpec((1, G3), lambda i: (0, 0)),
            pl.BlockSpec((1, G3), lambda i: (0, 0)),
            pl.BlockSpec((UP, NCLS), lambda i: (0, 0)),
            pl.BlockSpec((1, NCLS), lambda i: (0, 0)),
        ],
        out_specs=pl.BlockSpec((B, NCLS), lambda i: (0, 0)),
        out_shape=jax.ShapeDtypeStruct((B, NCLS), jnp.float32),
        scratch_shapes=[pltpu.VMEM((2, B, 6 * F), jnp.float32),
                        pltpu.VMEM((B, UP), jnp.float32)],
    )(x3, wb.astype(bf16), cbias, gk.astype(bf16), gr.astype(bf16), bi, br,
      dw.astype(bf16), db)
    return out
